# baseline (speedup 1.0000x reference)
import jax
import jax.numpy as jnp
from jax import lax
from jax.experimental import pallas as pl
from jax.experimental.pallas import tpu as pltpu
from jax.experimental.pallas import tpu_sc as plsc

VOCAB = 1000000
DIM = 64
B = 16384
ROWW = 128
NROW = VOCAB // 2

NC = 2
NS = 16
L = 16
NW = NC * NS
BPW = B // NW
CH = 128
NCH = BPW // CH

_LN2 = 0.6931471805599453
_SQRT2 = 1.4142135623730951


def _ln(x):
    bits = plsc.bitcast(x, jnp.int32)
    e = ((bits >> 23) & 0xFF) - 127
    m = plsc.bitcast((bits & 0x007FFFFF) | 0x3F800000, jnp.float32)
    big = m > _SQRT2
    m = jnp.where(big, m * 0.5, m)
    e = (e + jnp.where(big, 1, 0)).astype(jnp.float32)
    z = (m - 1.0) / (m + 1.0)
    z2 = z * z
    poly = 1.0 + z2 * (1.0 / 3.0 + z2 * (1.0 / 5.0 + z2 * (1.0 / 7.0 + z2 * (1.0 / 9.0))))
    return 2.0 * z * poly + e * _LN2


def _sc_body(cq_hbm, cs_hbm, xq_hbm, xs_hbm, lab_hbm, ctab_hbm, xtab_hbm,
             out_hbm, idx_cq, idx_cs, idx_xq, idx_xs, lab_v,
             rows_c, rows_x, out_v, sem_c, sem_x):
    wid = lax.axis_index("s") * NC + lax.axis_index("c")
    base = wid * BPW

    pltpu.sync_copy(cq_hbm.at[pl.ds(base, BPW)], idx_cq)
    pltpu.sync_copy(cs_hbm.at[pl.ds(base, BPW)], idx_cs)
    pltpu.sync_copy(xq_hbm.at[pl.ds(base, BPW)], idx_xq)
    pltpu.sync_copy(xs_hbm.at[pl.ds(base, BPW)], idx_xs)
    pltpu.sync_copy(lab_hbm.at[pl.ds(base, BPW)], lab_v)

    lane = lax.iota(jnp.int32, L)
    HALF = BPW // 2
    acc0 = jnp.zeros((L,), jnp.float32)

    for hf in range(2):
        hbase = hf * HALF
        copies = []
        for j in range(HALF // CH):
            copies.append(pltpu.make_async_copy(
                ctab_hbm.at[idx_cq.at[pl.ds(hbase + j * CH, CH)]],
                rows_c.at[pl.ds(j * CH, CH)], sem_c))
            copies.append(pltpu.make_async_copy(
                xtab_hbm.at[idx_xq.at[pl.ds(hbase + j * CH, CH)]],
                rows_x.at[pl.ds(j * CH, CH)], sem_x))
        for c in copies:
            c.start()
        for c in copies:
            c.wait()

        def bce_body(g, acc, hbase=hbase):
            base_r = g * L
            cs = idx_cs[pl.ds(hbase + base_r, L)]
            xs = idx_xs[pl.ds(hbase + base_r, L)]
            s = jnp.zeros((L,), jnp.float32)
            for r in range(L):
                i = base_r + r
                oc = cs[r]
                ox = xs[r]
                prod = rows_c[i, pl.ds(oc, L)] * rows_x[i, pl.ds(ox, L)]
                for k in range(1, DIM // L):
                    prod = (prod + rows_c[i, pl.ds(oc + k * L, L)]
                            * rows_x[i, pl.ds(ox + k * L, L)])
                for sh in (8, 4, 2, 1):
                    prod = prod + prod.at[lane ^ sh].get(mode="promise_in_bounds")
                s = jnp.where(lane == r, prod, s)
            y = lab_v[pl.ds(hbase + base_r, L)]
            p = 1.0 / (1.0 + jnp.exp(-s))
            loss = -(y * _ln(p + 1e-8) + (1.0 - y) * _ln((1.0 - p) + 1e-8))
            return acc + loss

        acc0 = lax.fori_loop(0, HALF // L, bce_body, acc0)

    out_v[...] = acc0
    pltpu.sync_copy(out_v, out_hbm.at[pl.ds(wid * L, L)])


@jax.jit
def _run(center_ids, context_ids, labels, center_table, context_weights):
    mesh = plsc.VectorSubcoreMesh(core_axis_name="c", subcore_axis_name="s")
    cid = center_ids.astype(jnp.int32)
    xid = context_ids.astype(jnp.int32)
    partials = pl.kernel(
        _sc_body,
        out_type=jax.ShapeDtypeStruct((NW * L,), jnp.float32),
        mesh=mesh,
        compiler_params=pltpu.CompilerParams(
            needs_layout_passes=False, use_tc_tiling_on_sc=True),
        scratch_types=[
            pltpu.VMEM((BPW,), jnp.int32),
            pltpu.VMEM((BPW,), jnp.int32),
            pltpu.VMEM((BPW,), jnp.int32),
            pltpu.VMEM((BPW,), jnp.int32),
            pltpu.VMEM((BPW,), jnp.float32),
            pltpu.VMEM((BPW // 2, ROWW), jnp.float32),
            pltpu.VMEM((BPW // 2, ROWW), jnp.float32),
            pltpu.VMEM((L,), jnp.float32),
            pltpu.SemaphoreType.DMA,
            pltpu.SemaphoreType.DMA,
        ],
    )(
        cid >> 1,
        (cid & 1) * DIM,
        xid >> 1,
        (xid & 1) * DIM,
        labels,
        center_table.reshape(NROW, ROWW),
        context_weights.reshape(NROW, ROWW),
    )
    return jnp.sum(partials) / B


def kernel(center_ids, context_ids, labels, center_table, context_weights):
    return _run(center_ids, context_ids, labels, center_table, context_weights)


# speedup vs baseline: 2.4113x; 2.4113x over previous
import jax
import jax.numpy as jnp
from jax import lax
from jax.experimental import pallas as pl
from jax.experimental.pallas import tpu as pltpu
from jax.experimental.pallas import tpu_sc as plsc

VOCAB = 1000000
DIM = 64
B = 16384
SUB = 8

NC = 2
NS = 16
L = 16
NW = NC * NS
BPW = B // NW
CH = 128
NCH = BPW // CH

_LN2 = 0.6931471805599453
_SQRT2 = 1.4142135623730951


def _ln(x):
    bits = plsc.bitcast(x, jnp.int32)
    e = ((bits >> 23) & 0xFF) - 127
    m = plsc.bitcast((bits & 0x007FFFFF) | 0x3F800000, jnp.float32)
    big = m > _SQRT2
    m = jnp.where(big, m * 0.5, m)
    e = (e + jnp.where(big, 1, 0)).astype(jnp.float32)
    z = (m - 1.0) / (m + 1.0)
    z2 = z * z
    poly = 1.0 + z2 * (1.0 / 3.0 + z2 * (1.0 / 5.0 + z2 * (1.0 / 7.0 + z2 * (1.0 / 9.0))))
    return 2.0 * z * poly + e * _LN2


NSEM = 4


def _sc_body(cq_hbm, cs_hbm, xq_hbm, xs_hbm, lab_hbm, ctab_hbm, xtab_hbm,
             out_hbm, idx_cq, idx_cs, idx_xq, idx_xs, lab_v,
             rows_c0, rows_x0, rows_c1, rows_x1, out_v, sems_c, sems_x):
    wid = lax.axis_index("s") * NC + lax.axis_index("c")
    base = wid * BPW

    ctab3 = ctab_hbm
    xtab3 = xtab_hbm
    bufs = ((rows_c0, rows_x0), (rows_c1, rows_x1))

    pltpu.sync_copy(cq_hbm.at[pl.ds(base, BPW)], idx_cq)
    pltpu.sync_copy(cs_hbm.at[pl.ds(base, BPW)], idx_cs)
    pltpu.sync_copy(xq_hbm.at[pl.ds(base, BPW)], idx_xq)
    pltpu.sync_copy(xs_hbm.at[pl.ds(base, BPW)], idx_xs)
    pltpu.sync_copy(lab_hbm.at[pl.ds(base, BPW)], lab_v)

    lane = lax.iota(jnp.int32, L)

    def fire_chunk(ch):
        b = ch % 2
        rc, rx = bufs[b]
        cbase = ch * CH

        def fire(g, carry):
            cq = idx_cq[pl.ds(cbase + g * L, L)]
            xq = idx_xq[pl.ds(cbase + g * L, L)]
            cs = idx_cs[pl.ds(cbase + g * L, L)]
            xs = idx_xs[pl.ds(cbase + g * L, L)]
            for r in range(L):
                i = g * L + r
                pltpu.make_async_copy(
                    ctab3.at[cq[r], cs[r]], rc.at[i // SUB, i % SUB],
                    sems_c.at[b]).start()
                pltpu.make_async_copy(
                    xtab3.at[xq[r], xs[r]], rx.at[i // SUB, i % SUB],
                    sems_x.at[b]).start()
            return carry

        lax.fori_loop(0, CH // L, fire, 0)

    def drain_chunk(ch):
        b = ch % 2
        rc, rx = bufs[b]
        pltpu.make_async_copy(ctab3.at[pl.ds(0, CH // SUB)], rc, sems_c.at[b]).wait()
        pltpu.make_async_copy(xtab3.at[pl.ds(0, CH // SUB)], rx, sems_x.at[b]).wait()

    def compute_chunk(ch, acc):
        rc, rx = bufs[ch % 2]
        cbase = ch * CH

        def bce_body(g, acc):
            s = jnp.zeros((L,), jnp.float32)
            for r in range(L):
                i = g * L + r
                a, sb = i // SUB, i % SUB
                prod = rc[a, sb, pl.ds(0, L)] * rx[a, sb, pl.ds(0, L)]
                for k in range(1, DIM // L):
                    prod = (prod + rc[a, sb, pl.ds(k * L, L)]
                            * rx[a, sb, pl.ds(k * L, L)])
                for sh in (8, 4, 2, 1):
                    prod = prod + prod.at[lane ^ sh].get(mode="promise_in_bounds")
                s = jnp.where(lane == r, prod, s)
            y = lab_v[pl.ds(cbase + g * L, L)]
            p = 1.0 / (1.0 + jnp.exp(-s))
            loss = -(y * _ln(p + 1e-8) + (1.0 - y) * _ln((1.0 - p) + 1e-8))
            return acc + loss

        return lax.fori_loop(0, CH // L, bce_body, acc)

    acc = jnp.zeros((L,), jnp.float32)
    fire_chunk(0)
    for ch in range(NCH):
        if ch + 1 < NCH:
            fire_chunk(ch + 1)
        drain_chunk(ch)
        acc = compute_chunk(ch, acc)

    out_v[...] = acc
    pltpu.sync_copy(out_v, out_hbm.at[pl.ds(wid * L, L)])


@jax.jit
def _run(center_ids, context_ids, labels, center_table, context_weights):
    mesh = plsc.VectorSubcoreMesh(core_axis_name="c", subcore_axis_name="s")
    cid = center_ids.astype(jnp.int32)
    xid = context_ids.astype(jnp.int32)
    partials = pl.kernel(
        _sc_body,
        out_type=jax.ShapeDtypeStruct((NW * L,), jnp.float32),
        mesh=mesh,
        compiler_params=pltpu.CompilerParams(
            needs_layout_passes=False, use_tc_tiling_on_sc=True),
        scratch_types=[
            pltpu.VMEM((BPW,), jnp.int32),
            pltpu.VMEM((BPW,), jnp.int32),
            pltpu.VMEM((BPW,), jnp.int32),
            pltpu.VMEM((BPW,), jnp.int32),
            pltpu.VMEM((BPW,), jnp.float32),
            pltpu.VMEM((CH // SUB, SUB, DIM), jnp.float32),
            pltpu.VMEM((CH // SUB, SUB, DIM), jnp.float32),
            pltpu.VMEM((CH // SUB, SUB, DIM), jnp.float32),
            pltpu.VMEM((CH // SUB, SUB, DIM), jnp.float32),
            pltpu.VMEM((L,), jnp.float32),
            pltpu.SemaphoreType.DMA((2,)),
            pltpu.SemaphoreType.DMA((2,)),
        ],
    )(
        cid >> 3,
        cid & 7,
        xid >> 3,
        xid & 7,
        labels,
        center_table.reshape(VOCAB // SUB, SUB, DIM),
        context_weights.reshape(VOCAB // SUB, SUB, DIM),
    )
    return jnp.sum(partials) / B


def kernel(center_ids, context_ids, labels, center_table, context_weights):
    return _run(center_ids, context_ids, labels, center_table, context_weights)


# speedup vs baseline: 2.4152x; 1.0016x over previous
import jax
import jax.numpy as jnp
from jax import lax
from jax.experimental import pallas as pl
from jax.experimental.pallas import tpu as pltpu
from jax.experimental.pallas import tpu_sc as plsc

VOCAB = 1000000
DIM = 64
B = 16384
SUB = 8

NC = 2
NS = 16
L = 16
NW = NC * NS
BPW = B // NW
CH = 128
NCH = BPW // CH

_LN2 = 0.6931471805599453
_SQRT2 = 1.4142135623730951


def _ln(x):
    bits = plsc.bitcast(x, jnp.int32)
    e = ((bits >> 23) & 0xFF) - 127
    m = plsc.bitcast((bits & 0x007FFFFF) | 0x3F800000, jnp.float32)
    big = m > _SQRT2
    m = jnp.where(big, m * 0.5, m)
    e = (e + jnp.where(big, 1, 0)).astype(jnp.float32)
    z = (m - 1.0) / (m + 1.0)
    z2 = z * z
    poly = 1.0 + z2 * (1.0 / 3.0 + z2 * (1.0 / 5.0 + z2 * (1.0 / 7.0 + z2 * (1.0 / 9.0))))
    return 2.0 * z * poly + e * _LN2


def _sc_body(cq_hbm, cs_hbm, xq_hbm, xs_hbm, lab_hbm, ctab_hbm, xtab_hbm,
             out_hbm, idx_cq, idx_cs, idx_xq, idx_xs, lab_v,
             rows_c0, rows_x0, rows_c1, rows_x1, out_v, sems_c, sems_x):
    wid = lax.axis_index("s") * NC + lax.axis_index("c")
    base = wid * BPW

    ctab3 = ctab_hbm
    xtab3 = xtab_hbm
    bufs = ((rows_c0, rows_x0), (rows_c1, rows_x1))

    pltpu.sync_copy(cq_hbm.at[pl.ds(base, BPW)], idx_cq)
    pltpu.sync_copy(cs_hbm.at[pl.ds(base, BPW)], idx_cs)
    pltpu.sync_copy(xq_hbm.at[pl.ds(base, BPW)], idx_xq)
    pltpu.sync_copy(xs_hbm.at[pl.ds(base, BPW)], idx_xs)
    pltpu.sync_copy(lab_hbm.at[pl.ds(base, BPW)], lab_v)

    lane = lax.iota(jnp.int32, L)

    def fire_chunk(ch):
        b = ch % 2
        rc, rx = bufs[b]
        cbase = ch * CH

        def fire(g, carry):
            cq = idx_cq[pl.ds(cbase + g * L, L)]
            xq = idx_xq[pl.ds(cbase + g * L, L)]
            cs = idx_cs[pl.ds(cbase + g * L, L)]
            xs = idx_xs[pl.ds(cbase + g * L, L)]
            for r in range(L):
                i = g * L + r
                pltpu.make_async_copy(
                    ctab3.at[cq[r], cs[r]], rc.at[i // SUB, i % SUB],
                    sems_c.at[b]).start()
                pltpu.make_async_copy(
                    xtab3.at[xq[r], xs[r]], rx.at[i // SUB, i % SUB],
                    sems_x.at[b]).start()
            return carry

        lax.fori_loop(0, CH // L, fire, 0)

    def drain_chunk(ch):
        b = ch % 2
        rc, rx = bufs[b]
        pltpu.make_async_copy(ctab3.at[pl.ds(0, CH // SUB)], rc, sems_c.at[b]).wait()
        pltpu.make_async_copy(xtab3.at[pl.ds(0, CH // SUB)], rx, sems_x.at[b]).wait()

    def compute_chunk(ch, acc):
        rc, rx = bufs[ch % 2]
        cbase = ch * CH

        def bce_body(g, acc):
            s = jnp.zeros((L,), jnp.float32)
            for r in range(L):
                i = g * L + r
                a, sb = i // SUB, i % SUB
                prod = rc[a, sb, pl.ds(0, L)] * rx[a, sb, pl.ds(0, L)]
                for k in range(1, DIM // L):
                    prod = (prod + rc[a, sb, pl.ds(k * L, L)]
                            * rx[a, sb, pl.ds(k * L, L)])
                for sh in (8, 4, 2, 1):
                    prod = prod + prod.at[lane ^ sh].get(mode="promise_in_bounds")
                s = jnp.where(lane == r, prod, s)
            y = lab_v[pl.ds(cbase + g * L, L)]
            p = 1.0 / (1.0 + jnp.exp(-s))
            loss = -(y * _ln(p + 1e-8) + (1.0 - y) * _ln((1.0 - p) + 1e-8))
            return acc + loss

        return lax.fori_loop(0, CH // L, bce_body, acc)

    acc = jnp.zeros((L,), jnp.float32)
    fire_chunk(0)
    for ch in range(NCH):
        if ch + 1 < NCH:
            fire_chunk(ch + 1)
        drain_chunk(ch)
        acc = compute_chunk(ch, acc)

    out_v[...] = acc
    pltpu.sync_copy(out_v, out_hbm.at[pl.ds(wid * L, L)])


@jax.jit
def _run(center_ids, context_ids, labels, center_table, context_weights):
    mesh = plsc.VectorSubcoreMesh(core_axis_name="c", subcore_axis_name="s")
    cid = center_ids.astype(jnp.int32)
    xid = context_ids.astype(jnp.int32)
    partials = pl.kernel(
        _sc_body,
        out_type=jax.ShapeDtypeStruct((NW * L,), jnp.float32),
        mesh=mesh,
        compiler_params=pltpu.CompilerParams(
            needs_layout_passes=False, use_tc_tiling_on_sc=True),
        scratch_types=[
            pltpu.VMEM((BPW,), jnp.int32),
            pltpu.VMEM((BPW,), jnp.int32),
            pltpu.VMEM((BPW,), jnp.int32),
            pltpu.VMEM((BPW,), jnp.int32),
            pltpu.VMEM((BPW,), jnp.float32),
            pltpu.VMEM((CH // SUB, SUB, DIM), jnp.float32),
            pltpu.VMEM((CH // SUB, SUB, DIM), jnp.float32),
            pltpu.VMEM((CH // SUB, SUB, DIM), jnp.float32),
            pltpu.VMEM((CH // SUB, SUB, DIM), jnp.float32),
            pltpu.VMEM((L,), jnp.float32),
            pltpu.SemaphoreType.DMA((2,)),
            pltpu.SemaphoreType.DMA((2,)),
        ],
    )(
        cid >> 3,
        cid & 7,
        xid >> 3,
        xid & 7,
        labels,
        center_table.reshape(VOCAB // SUB, SUB, DIM),
        context_weights.reshape(VOCAB // SUB, SUB, DIM),
    )
    return jnp.sum(partials) / B


def kernel(center_ids, context_ids, labels, center_table, context_weights):
    return _run(center_ids, context_ids, labels, center_table, context_weights)


# speedup vs baseline: 2.4225x; 1.0030x over previous
import jax
import jax.numpy as jnp
from jax import lax
from jax.experimental import pallas as pl
from jax.experimental.pallas import tpu as pltpu
from jax.experimental.pallas import tpu_sc as plsc

VOCAB = 1000000
DIM = 64
B = 16384
SUB = 8

NC = 2
NS = 16
L = 16
NW = NC * NS
BPW = B // NW
CH = 128
NCH = BPW // CH

_LN2 = 0.6931471805599453
_SQRT2 = 1.4142135623730951


def _ln(x):
    bits = plsc.bitcast(x, jnp.int32)
    e = ((bits >> 23) & 0xFF) - 127
    m = plsc.bitcast((bits & 0x007FFFFF) | 0x3F800000, jnp.float32)
    big = m > _SQRT2
    m = jnp.where(big, m * 0.5, m)
    e = (e + jnp.where(big, 1, 0)).astype(jnp.float32)
    z = (m - 1.0) / (m + 1.0)
    z2 = z * z
    poly = 1.0 + z2 * (1.0 / 3.0 + z2 * (1.0 / 5.0 + z2 * (1.0 / 7.0 + z2 * (1.0 / 9.0))))
    return 2.0 * z * poly + e * _LN2


def _sc_body(cq_hbm, cs_hbm, xq_hbm, xs_hbm, lab_hbm, ctab_hbm, xtab_hbm,
             out_hbm, idx_cq, idx_cs, idx_xq, idx_xs, lab_v,
             rows_c0, rows_x0, rows_c1, rows_x1, out_v, sems_c, sems_x):
    wid = lax.axis_index("s") * NC + lax.axis_index("c")
    base = wid * BPW

    ctab3 = ctab_hbm
    xtab3 = xtab_hbm
    bufs = ((rows_c0, rows_x0), (rows_c1, rows_x1))

    staging = [
        pltpu.make_async_copy(cq_hbm.at[pl.ds(base, BPW)], idx_cq, sems_c.at[0]),
        pltpu.make_async_copy(cs_hbm.at[pl.ds(base, BPW)], idx_cs, sems_c.at[0]),
        pltpu.make_async_copy(xq_hbm.at[pl.ds(base, BPW)], idx_xq, sems_c.at[0]),
        pltpu.make_async_copy(xs_hbm.at[pl.ds(base, BPW)], idx_xs, sems_c.at[0]),
        pltpu.make_async_copy(lab_hbm.at[pl.ds(base, BPW)], lab_v, sems_c.at[0]),
    ]
    for c in staging:
        c.start()
    for c in staging:
        c.wait()

    lane = lax.iota(jnp.int32, L)

    def fire_chunk(ch):
        b = ch % 2
        rc, rx = bufs[b]
        cbase = ch * CH

        def fire(g, carry):
            cq = idx_cq[pl.ds(cbase + g * L, L)]
            xq = idx_xq[pl.ds(cbase + g * L, L)]
            cs = idx_cs[pl.ds(cbase + g * L, L)]
            xs = idx_xs[pl.ds(cbase + g * L, L)]
            for r in range(L):
                i = g * L + r
                pltpu.make_async_copy(
                    ctab3.at[cq[r], cs[r]], rc.at[i // SUB, i % SUB],
                    sems_c.at[b]).start()
                pltpu.make_async_copy(
                    xtab3.at[xq[r], xs[r]], rx.at[i // SUB, i % SUB],
                    sems_x.at[b]).start()
            return carry

        lax.fori_loop(0, CH // L, fire, 0)

    def drain_chunk(ch):
        b = ch % 2
        rc, rx = bufs[b]
        pltpu.make_async_copy(ctab3.at[pl.ds(0, CH // SUB)], rc, sems_c.at[b]).wait()
        pltpu.make_async_copy(xtab3.at[pl.ds(0, CH // SUB)], rx, sems_x.at[b]).wait()

    def compute_chunk(ch, acc):
        rc, rx = bufs[ch % 2]
        cbase = ch * CH

        def bce_body(g, acc):
            s = jnp.zeros((L,), jnp.float32)
            for r in range(L):
                i = g * L + r
                a, sb = i // SUB, i % SUB
                prod = rc[a, sb, pl.ds(0, L)] * rx[a, sb, pl.ds(0, L)]
                for k in range(1, DIM // L):
                    prod = (prod + rc[a, sb, pl.ds(k * L, L)]
                            * rx[a, sb, pl.ds(k * L, L)])
                for sh in (8, 4, 2, 1):
                    prod = prod + prod.at[lane ^ sh].get(mode="promise_in_bounds")
                s = jnp.where(lane == r, prod, s)
            y = lab_v[pl.ds(cbase + g * L, L)]
            p = 1.0 / (1.0 + jnp.exp(-s))
            loss = -(y * _ln(p + 1e-8) + (1.0 - y) * _ln((1.0 - p) + 1e-8))
            return acc + loss

        return lax.fori_loop(0, CH // L, bce_body, acc)

    acc = jnp.zeros((L,), jnp.float32)
    fire_chunk(0)
    for ch in range(NCH):
        if ch + 1 < NCH:
            fire_chunk(ch + 1)
        drain_chunk(ch)
        acc = compute_chunk(ch, acc)

    out_v[...] = acc
    pltpu.sync_copy(out_v, out_hbm.at[pl.ds(wid * L, L)])


@jax.jit
def _run(center_ids, context_ids, labels, center_table, context_weights):
    mesh = plsc.VectorSubcoreMesh(core_axis_name="c", subcore_axis_name="s")
    cid = center_ids.astype(jnp.int32)
    xid = context_ids.astype(jnp.int32)
    partials = pl.kernel(
        _sc_body,
        out_type=jax.ShapeDtypeStruct((NW * L,), jnp.float32),
        mesh=mesh,
        compiler_params=pltpu.CompilerParams(
            needs_layout_passes=False, use_tc_tiling_on_sc=True),
        scratch_types=[
            pltpu.VMEM((BPW,), jnp.int32),
            pltpu.VMEM((BPW,), jnp.int32),
            pltpu.VMEM((BPW,), jnp.int32),
            pltpu.VMEM((BPW,), jnp.int32),
            pltpu.VMEM((BPW,), jnp.float32),
            pltpu.VMEM((CH // SUB, SUB, DIM), jnp.float32),
            pltpu.VMEM((CH // SUB, SUB, DIM), jnp.float32),
            pltpu.VMEM((CH // SUB, SUB, DIM), jnp.float32),
            pltpu.VMEM((CH // SUB, SUB, DIM), jnp.float32),
            pltpu.VMEM((L,), jnp.float32),
            pltpu.SemaphoreType.DMA((2,)),
            pltpu.SemaphoreType.DMA((2,)),
        ],
    )(
        cid >> 3,
        cid & 7,
        xid >> 3,
        xid & 7,
        labels,
        center_table.reshape(VOCAB // SUB, SUB, DIM),
        context_weights.reshape(VOCAB // SUB, SUB, DIM),
    )
    return jnp.sum(partials) / B


def kernel(center_ids, context_ids, labels, center_table, context_weights):
    return _run(center_ids, context_ids, labels, center_table, context_weights)
